# Initial kernel scaffold; baseline (speedup 1.0000x reference)
import jax
import jax.numpy as jnp
from jax.experimental import pallas as pl


def kernel(x, edge_idx, W1, b1, W2, b2, Wd, bd):
    raise NotImplementedError("write your pallas kernel here")


# speedup vs baseline: 13.8835x; 13.8835x over previous
import functools

import jax
import jax.numpy as jnp
from jax import lax
from jax.experimental import pallas as pl
from jax.experimental.pallas import tpu as pltpu
from jax.experimental.pallas import tpu_sc as plsc

NC = 2
NS = 16
L = 16


def _npad(N, R):
    NP = ((N + R - 1) // R) * R
    assert NP % (NS * 8) == 0
    return NP


@functools.lru_cache(maxsize=None)
def _make_count(NP, E):
    NW = NC * NS
    C = 80
    assert E % (NW * C) == 0
    EPW = E // NW
    NCHUNK = EPW // C
    RPT = NP // NS
    ZR = RPT // 5
    assert RPT % ZR == 0 and ZR >= C

    mesh = plsc.VectorSubcoreMesh(core_axis_name="c", subcore_axis_name="s")

    @functools.partial(
        pl.kernel,
        mesh=mesh,
        out_type=jax.ShapeDtypeStruct((NC, NP, L), jnp.float32),
        scratch_types=[
            pltpu.VMEM((C,), jnp.int32),
            pltpu.VMEM((C, L), jnp.float32),
            pltpu.VMEM((ZR, L), jnp.float32),
            pltpu.VMEM_SHARED((NP, L), jnp.float32),
        ],
        compiler_params=pltpu.CompilerParams(use_tc_tiling_on_sc=False),
    )
    def count_kernel(dst_hbm, out_hbm, idx_v, ones_v, zbuf, acc_sh):
        cid = lax.axis_index("c")
        sid = lax.axis_index("s")
        wid = sid * NC + cid

        def fill(r, carry):
            zbuf[r, :] = jnp.zeros((L,), jnp.float32)
            return carry

        lax.fori_loop(0, ZR, fill, 0)

        def fill_ones(r, carry):
            ones_v[r, :] = jnp.ones((L,), jnp.float32)
            return carry

        lax.fori_loop(0, C, fill_ones, 0)

        for z in range(RPT // ZR):
            pltpu.sync_copy(zbuf, acc_sh.at[pl.ds(sid * RPT + z * ZR, ZR)])
        plsc.subcore_barrier()

        def step(j, carry):
            base = wid * EPW + j * C
            pltpu.sync_copy(dst_hbm.at[pl.ds(base, C)], idx_v)
            pltpu.sync_copy(ones_v, acc_sh.at[idx_v], add=True)
            return carry

        lax.fori_loop(0, NCHUNK, step, 0)
        plsc.subcore_barrier()

        pltpu.sync_copy(acc_sh.at[pl.ds(sid * RPT, RPT)],
                        out_hbm.at[cid, pl.ds(sid * RPT, RPT)])

    return count_kernel


@functools.lru_cache(maxsize=None)
def _make_agg(NP, E, D):
    NW = NC * NS
    C = 80
    assert E % (NW * C) == 0
    EPW = E // NW
    NCHUNK = EPW // C
    RPT = NP // NS
    ZR = RPT // 5
    assert RPT % ZR == 0

    mesh = plsc.VectorSubcoreMesh(core_axis_name="c", subcore_axis_name="s")

    @functools.partial(
        pl.kernel,
        mesh=mesh,
        out_type=jax.ShapeDtypeStruct((NC, NP, D), jnp.float32),
        scratch_types=[
            pltpu.VMEM((C,), jnp.int32),
            pltpu.VMEM((C,), jnp.int32),
            pltpu.VMEM((C, D), jnp.float32),
            pltpu.VMEM((ZR, D), jnp.float32),
            pltpu.VMEM_SHARED((NP, D), jnp.float32),
            pltpu.SemaphoreType.DMA,
        ],
        compiler_params=pltpu.CompilerParams(use_tc_tiling_on_sc=False),
    )
    def agg_kernel(hs_hbm, src_hbm, dst_hbm, out_hbm,
                   sidx, didx, rows, zbuf, acc_sh, sem):
        cid = lax.axis_index("c")
        sid = lax.axis_index("s")
        wid = sid * NC + cid

        def fill(r, carry):
            for cc in range(D // L):
                zbuf[r, pl.ds(cc * L, L)] = jnp.zeros((L,), jnp.float32)
            return carry

        lax.fori_loop(0, ZR, fill, 0)

        for z in range(RPT // ZR):
            pltpu.sync_copy(zbuf, acc_sh.at[pl.ds(sid * RPT + z * ZR, ZR)])
        plsc.subcore_barrier()

        def step(j, carry):
            base = wid * EPW + j * C
            pltpu.sync_copy(src_hbm.at[pl.ds(base, C)], sidx)
            pltpu.sync_copy(dst_hbm.at[pl.ds(base, C)], didx)
            pltpu.async_copy(hs_hbm.at[sidx], rows, sem).wait()
            pltpu.sync_copy(rows, acc_sh.at[didx], add=True)
            return carry

        lax.fori_loop(0, NCHUNK, step, 0)
        plsc.subcore_barrier()

        pltpu.sync_copy(acc_sh.at[pl.ds(sid * RPT, RPT)],
                        out_hbm.at[cid, pl.ds(sid * RPT, RPT)])

    return agg_kernel


def _dinv_block(cnt_ref):
    cnt = cnt_ref[0]
    deg = cnt[0] + cnt[1] + 1.0
    return lax.rsqrt(deg)


def _leaky(z):
    return jnp.where(z >= 0, z, 0.01 * z)


def _cnt_spec(R):
    return pl.BlockSpec((1, 2, R), lambda i: (i, 0, 0))


def _tc_pre_body(x_ref, w1_ref, wd_ref, bd_ref, cnt_ref, hs1_ref, ident_ref):
    dinv = _dinv_block(cnt_ref)
    x = x_ref[...]
    h1 = jnp.dot(x, w1_ref[...], preferred_element_type=jnp.float32)
    hs1_ref[...] = h1 * dinv[:, None]
    ident_ref[...] = (
        jnp.dot(x, wd_ref[...], preferred_element_type=jnp.float32)
        + bd_ref[...]
    )


def _tc_mid_body(agg_ref, hs1_ref, cnt_ref, b1_ref, w2_ref, hs2_ref):
    dinv = _dinv_block(cnt_ref)
    a = agg_ref[0] + agg_ref[1] + hs1_ref[...]
    o1 = _leaky(a * dinv[:, None] + b1_ref[...])
    h2 = jnp.dot(o1, w2_ref[...], preferred_element_type=jnp.float32)
    hs2_ref[...] = h2 * dinv[:, None]


def _tc_post_body(agg_ref, hs2_ref, cnt_ref, b2_ref, ident_ref, out_ref):
    dinv = _dinv_block(cnt_ref)
    a = agg_ref[0] + agg_ref[1] + hs2_ref[...]
    o2 = _leaky(a * dinv[:, None] + b2_ref[...])
    out_ref[...] = o2 + ident_ref[...]


def _row_spec(R, D):
    return pl.BlockSpec((R, D), lambda i: (i, 0))


def _part_spec(R, D):
    return pl.BlockSpec((2, R, D), lambda i: (0, i, 0))


def _full_spec(shape):
    nd = len(shape)
    return pl.BlockSpec(shape, lambda i: (0,) * nd)


def _tc_pre(x, W1, Wd, bd, cnt, R):
    NP, F = x.shape
    H = W1.shape[1]
    O = Wd.shape[1]
    grid = NP // R
    return pl.pallas_call(
        _tc_pre_body,
        grid=(grid,),
        in_specs=[
            _row_spec(R, F),
            _full_spec(W1.shape),
            _full_spec(Wd.shape),
            _full_spec((1, O)),
            _cnt_spec(R),
        ],
        out_specs=[_row_spec(R, H), _row_spec(R, O)],
        out_shape=[
            jax.ShapeDtypeStruct((NP, H), jnp.float32),
            jax.ShapeDtypeStruct((NP, O), jnp.float32),
        ],
    )(x, W1, Wd, bd.reshape(1, O), cnt)


def _tc_mid(agg1, hs1, cnt, b1, W2, R):
    NP, H = hs1.shape
    O = W2.shape[1]
    grid = NP // R
    return pl.pallas_call(
        _tc_mid_body,
        grid=(grid,),
        in_specs=[
            _part_spec(R, H),
            _row_spec(R, H),
            _cnt_spec(R),
            _full_spec((1, H)),
            _full_spec(W2.shape),
        ],
        out_specs=_row_spec(R, O),
        out_shape=jax.ShapeDtypeStruct((NP, O), jnp.float32),
    )(agg1, hs1, cnt, b1.reshape(1, H), W2)


def _tc_post(agg2, hs2, cnt, b2, ident, R):
    NP, O = hs2.shape
    grid = NP // R
    return pl.pallas_call(
        _tc_post_body,
        grid=(grid,),
        in_specs=[
            _part_spec(R, O),
            _row_spec(R, O),
            _cnt_spec(R),
            _full_spec((1, O)),
            _row_spec(R, O),
        ],
        out_specs=_row_spec(R, O),
        out_shape=jax.ShapeDtypeStruct((NP, O), jnp.float32),
    )(agg2, hs2, cnt, b2.reshape(1, O), ident)


@jax.jit
def kernel(x, edge_idx, W1, b1, W2, b2, Wd, bd):
    N, F = x.shape
    E = edge_idx.shape[1]
    H = W1.shape[1]
    O = W2.shape[1]
    R = 2048
    NP = _npad(N, R)

    src = edge_idx[0]
    dst = edge_idx[1]
    xp = jnp.zeros((NP, F), x.dtype).at[:N].set(x)

    cnt_parts = _make_count(NP, E)(dst)
    cnt = cnt_parts[:, :, 0]
    cnt = cnt.reshape(2, NP // R, R).transpose(1, 0, 2)

    hs1, ident = _tc_pre(xp, W1, Wd, bd, cnt, R)
    agg1 = _make_agg(NP, E, H)(hs1, src, dst)
    hs2 = _tc_mid(agg1, hs1, cnt, b1, W2, R)
    agg2 = _make_agg(NP, E, O)(hs2, src, dst)
    out = _tc_post(agg2, hs2, cnt, b2, ident, R)
    return out[:N]


# speedup vs baseline: 35.3782x; 2.5482x over previous
import functools

import jax
import jax.numpy as jnp
from jax import lax
from jax.experimental import pallas as pl
from jax.experimental.pallas import tpu as pltpu
from jax.experimental.pallas import tpu_sc as plsc

NC = 2
NS = 16
L = 16


def _npad(N, R):
    NP = ((N + R - 1) // R) * R
    assert NP % (NS * 8) == 0
    return NP


CNT_C = 80


@functools.lru_cache(maxsize=None)
def _make_count(NP, E):
    NW = NC * NS
    C = CNT_C
    assert E % (NW * C) == 0
    EPW = E // NW
    NCHUNK = EPW // C
    RPT = NP // NS
    ZR = RPT // 5
    assert RPT % ZR == 0 and ZR >= C

    mesh = plsc.VectorSubcoreMesh(core_axis_name="c", subcore_axis_name="s")

    SLAG = 4

    @functools.partial(
        pl.kernel,
        mesh=mesh,
        out_type=jax.ShapeDtypeStruct((NC, NP, L), jnp.float32),
        scratch_types=[
            pltpu.VMEM((NCHUNK, C), jnp.int32),
            pltpu.VMEM((C, L), jnp.float32),
            pltpu.VMEM((ZR, L), jnp.float32),
            pltpu.VMEM_SHARED((NP, L), jnp.float32),
            pltpu.SemaphoreType.DMA,
        ],
        compiler_params=pltpu.CompilerParams(use_tc_tiling_on_sc=False),
    )
    def count_kernel(dst_hbm, out_hbm, didx_all, ones_v, zbuf, acc_sh, ssem):
        cid = lax.axis_index("c")
        sid = lax.axis_index("s")
        wid = sid * NC + cid

        def fill(r, carry):
            zbuf[r, :] = jnp.zeros((L,), jnp.float32)
            return carry

        lax.fori_loop(0, ZR, fill, 0)

        def fill_ones(r, carry):
            ones_v[r, :] = jnp.ones((L,), jnp.float32)
            return carry

        lax.fori_loop(0, C, fill_ones, 0)

        pltpu.sync_copy(dst_hbm.at[wid], didx_all)
        for z in range(RPT // ZR):
            pltpu.sync_copy(zbuf, acc_sh.at[pl.ds(sid * RPT + z * ZR, ZR)])
        plsc.subcore_barrier()

        def drain_one():
            pltpu.make_async_copy(
                ones_v, acc_sh.at[didx_all.at[0]], ssem).wait()

        def step(j, carry):
            pltpu.async_copy(ones_v, acc_sh.at[didx_all.at[j]], ssem,
                             add=True)

            @pl.when(j >= SLAG)
            def _():
                drain_one()

            return carry

        lax.fori_loop(0, NCHUNK, step, 0)
        for _ in range(SLAG):
            drain_one()
        plsc.subcore_barrier()

        pltpu.sync_copy(acc_sh.at[pl.ds(sid * RPT, RPT)],
                        out_hbm.at[cid, pl.ds(sid * RPT, RPT)])

    return count_kernel


AGG_C = 40


@functools.lru_cache(maxsize=None)
def _make_agg(NP, E, D):
    NW = NC * NS
    C = AGG_C
    assert E % (NW * C) == 0
    EPW = E // NW
    NCHUNK = EPW // C
    RPT = NP // NS
    assert RPT % C == 0

    mesh = plsc.VectorSubcoreMesh(core_axis_name="c", subcore_axis_name="s")

    P = 5
    G = 2
    assert NCHUNK % P == 0 and G < P

    @functools.partial(
        pl.kernel,
        mesh=mesh,
        out_type=jax.ShapeDtypeStruct((NC, NP, D), jnp.float32),
        scratch_types=[
            pltpu.VMEM((NCHUNK, C), jnp.int32),
            pltpu.VMEM((NCHUNK, C), jnp.int32),
            pltpu.VMEM((P, C, D), jnp.float32),
            pltpu.VMEM_SHARED((NP, D), jnp.float32),
            pltpu.SemaphoreType.DMA,
            pltpu.SemaphoreType.DMA,
        ],
        compiler_params=pltpu.CompilerParams(use_tc_tiling_on_sc=False),
    )
    def agg_kernel(hs_hbm, src_hbm, dst_hbm, out_hbm,
                   sidx_all, didx_all, rows, acc_sh, gsem, ssem):
        cid = lax.axis_index("c")
        sid = lax.axis_index("s")
        wid = sid * NC + cid

        def fill(r, carry):
            for cc in range(D // L):
                rows[0, r, pl.ds(cc * L, L)] = jnp.zeros((L,), jnp.float32)
            return carry

        lax.fori_loop(0, C, fill, 0)

        pltpu.sync_copy(src_hbm.at[wid], sidx_all)
        pltpu.sync_copy(dst_hbm.at[wid], didx_all)
        for z in range(RPT // C):
            pltpu.sync_copy(rows.at[0],
                            acc_sh.at[pl.ds(sid * RPT + z * C, C)])
        plsc.subcore_barrier()

        def issue_gather(j, b):
            pltpu.async_copy(hs_hbm.at[sidx_all.at[j]], rows.at[b], gsem)

        def drain_gather():
            pltpu.make_async_copy(
                hs_hbm.at[sidx_all.at[0]], rows.at[0], gsem).wait()

        def drain_scatter():
            pltpu.make_async_copy(
                rows.at[0], acc_sh.at[didx_all.at[0]], ssem).wait()

        for b in range(G):
            issue_gather(b, b)

        def turn(g, carry):
            for b in range(P):
                t = g * P + b

                @pl.when(t >= G)
                def _():
                    drain_scatter()

                @pl.when(t + G < NCHUNK)
                def _():
                    issue_gather(t + G, (b + G) % P)

                drain_gather()
                pltpu.async_copy(rows.at[b], acc_sh.at[didx_all.at[t]],
                                 ssem, add=True)
            return carry

        lax.fori_loop(0, NCHUNK // P, turn, 0)
        for _ in range(G):
            drain_scatter()
        plsc.subcore_barrier()

        pltpu.sync_copy(acc_sh.at[pl.ds(sid * RPT, RPT)],
                        out_hbm.at[cid, pl.ds(sid * RPT, RPT)])

    return agg_kernel


def _dinv_block(cnt_ref):
    cnt = cnt_ref[0]
    deg = cnt[0] + cnt[1] + 1.0
    return lax.rsqrt(deg)


def _leaky(z):
    return jnp.where(z >= 0, z, 0.01 * z)


def _cnt_spec(R):
    return pl.BlockSpec((1, 2, R), lambda i: (i, 0, 0))


def _tc_pre_body(x_ref, w1_ref, wd_ref, bd_ref, cnt_ref, hs1_ref, ident_ref):
    dinv = _dinv_block(cnt_ref)
    x = x_ref[...]
    h1 = jnp.dot(x, w1_ref[...], preferred_element_type=jnp.float32)
    hs1_ref[...] = h1 * dinv[:, None]
    ident_ref[...] = (
        jnp.dot(x, wd_ref[...], preferred_element_type=jnp.float32)
        + bd_ref[...]
    )


def _tc_mid_body(agg_ref, hs1_ref, cnt_ref, b1_ref, w2_ref, hs2_ref):
    dinv = _dinv_block(cnt_ref)
    a = agg_ref[0] + agg_ref[1] + hs1_ref[...]
    o1 = _leaky(a * dinv[:, None] + b1_ref[...])
    h2 = jnp.dot(o1, w2_ref[...], preferred_element_type=jnp.float32)
    hs2_ref[...] = h2 * dinv[:, None]


def _tc_post_body(agg_ref, hs2_ref, cnt_ref, b2_ref, ident_ref, out_ref):
    dinv = _dinv_block(cnt_ref)
    a = agg_ref[0] + agg_ref[1] + hs2_ref[...]
    o2 = _leaky(a * dinv[:, None] + b2_ref[...])
    out_ref[...] = o2 + ident_ref[...]


def _row_spec(R, D):
    return pl.BlockSpec((R, D), lambda i: (i, 0))


def _part_spec(R, D):
    return pl.BlockSpec((2, R, D), lambda i: (0, i, 0))


def _full_spec(shape):
    nd = len(shape)
    return pl.BlockSpec(shape, lambda i: (0,) * nd)


def _tc_pre(x, W1, Wd, bd, cnt, R):
    NP, F = x.shape
    H = W1.shape[1]
    O = Wd.shape[1]
    grid = NP // R
    return pl.pallas_call(
        _tc_pre_body,
        grid=(grid,),
        in_specs=[
            _row_spec(R, F),
            _full_spec(W1.shape),
            _full_spec(Wd.shape),
            _full_spec((1, O)),
            _cnt_spec(R),
        ],
        out_specs=[_row_spec(R, H), _row_spec(R, O)],
        out_shape=[
            jax.ShapeDtypeStruct((NP, H), jnp.float32),
            jax.ShapeDtypeStruct((NP, O), jnp.float32),
        ],
    )(x, W1, Wd, bd.reshape(1, O), cnt)


def _tc_mid(agg1, hs1, cnt, b1, W2, R):
    NP, H = hs1.shape
    O = W2.shape[1]
    grid = NP // R
    return pl.pallas_call(
        _tc_mid_body,
        grid=(grid,),
        in_specs=[
            _part_spec(R, H),
            _row_spec(R, H),
            _cnt_spec(R),
            _full_spec((1, H)),
            _full_spec(W2.shape),
        ],
        out_specs=_row_spec(R, O),
        out_shape=jax.ShapeDtypeStruct((NP, O), jnp.float32),
    )(agg1, hs1, cnt, b1.reshape(1, H), W2)


def _tc_post(agg2, hs2, cnt, b2, ident, R):
    NP, O = hs2.shape
    grid = NP // R
    return pl.pallas_call(
        _tc_post_body,
        grid=(grid,),
        in_specs=[
            _part_spec(R, O),
            _row_spec(R, O),
            _cnt_spec(R),
            _full_spec((1, O)),
            _row_spec(R, O),
        ],
        out_specs=_row_spec(R, O),
        out_shape=jax.ShapeDtypeStruct((NP, O), jnp.float32),
    )(agg2, hs2, cnt, b2.reshape(1, O), ident)


@jax.jit
def kernel(x, edge_idx, W1, b1, W2, b2, Wd, bd):
    N, F = x.shape
    E = edge_idx.shape[1]
    H = W1.shape[1]
    O = W2.shape[1]
    R = 2048
    NP = _npad(N, R)

    NW = NC * NS
    src_r = edge_idx[0].reshape(NW, E // (NW * AGG_C), AGG_C)
    dst_r = edge_idx[1].reshape(NW, E // (NW * AGG_C), AGG_C)
    dst_c = edge_idx[1].reshape(NW, E // (NW * CNT_C), CNT_C)
    xp = jnp.zeros((NP, F), x.dtype).at[:N].set(x)

    cnt_parts = _make_count(NP, E)(dst_c)
    cnt = cnt_parts[:, :, 0]
    cnt = cnt.reshape(2, NP // R, R).transpose(1, 0, 2)

    hs1, ident = _tc_pre(xp, W1, Wd, bd, cnt, R)
    agg1 = _make_agg(NP, E, H)(hs1, src_r, dst_r)
    hs2 = _tc_mid(agg1, hs1, cnt, b1, W2, R)
    agg2 = _make_agg(NP, E, O)(hs2, src_r, dst_r)
    out = _tc_post(agg2, hs2, cnt, b2, ident, R)
    return out[:N]


# speedup vs baseline: 35.6740x; 1.0084x over previous
import functools

import jax
import jax.numpy as jnp
from jax import lax
from jax.experimental import pallas as pl
from jax.experimental.pallas import tpu as pltpu
from jax.experimental.pallas import tpu_sc as plsc

NC = 2
NS = 16
L = 16


def _npad(N, R):
    NP = ((N + R - 1) // R) * R
    assert NP % (NS * 8) == 0
    return NP


CNT_C = 80


@functools.lru_cache(maxsize=None)
def _make_count(NP, E):
    NW = NC * NS
    C = CNT_C
    assert E % (NW * C) == 0
    EPW = E // NW
    NCHUNK = EPW // C
    RPT = NP // NS
    ZR = RPT // 5
    assert RPT % ZR == 0 and ZR >= C

    mesh = plsc.VectorSubcoreMesh(core_axis_name="c", subcore_axis_name="s")

    SLAG = 4

    @functools.partial(
        pl.kernel,
        mesh=mesh,
        out_type=jax.ShapeDtypeStruct((NC, NP, L), jnp.float32),
        scratch_types=[
            pltpu.VMEM((NCHUNK, C), jnp.int32),
            pltpu.VMEM((C, L), jnp.float32),
            pltpu.VMEM((ZR, L), jnp.float32),
            pltpu.VMEM_SHARED((NP, L), jnp.float32),
            pltpu.SemaphoreType.DMA,
        ],
        compiler_params=pltpu.CompilerParams(use_tc_tiling_on_sc=False),
    )
    def count_kernel(dst_hbm, out_hbm, didx_all, ones_v, zbuf, acc_sh, ssem):
        cid = lax.axis_index("c")
        sid = lax.axis_index("s")
        wid = sid * NC + cid

        def fill(r, carry):
            zbuf[r, :] = jnp.zeros((L,), jnp.float32)
            return carry

        lax.fori_loop(0, ZR, fill, 0)

        def fill_ones(r, carry):
            ones_v[r, :] = jnp.ones((L,), jnp.float32)
            return carry

        lax.fori_loop(0, C, fill_ones, 0)

        pltpu.sync_copy(dst_hbm.at[wid], didx_all)
        for z in range(RPT // ZR):
            pltpu.sync_copy(zbuf, acc_sh.at[pl.ds(sid * RPT + z * ZR, ZR)])
        plsc.subcore_barrier()

        def drain_one():
            pltpu.make_async_copy(
                ones_v, acc_sh.at[didx_all.at[0]], ssem).wait()

        def step(j, carry):
            pltpu.async_copy(ones_v, acc_sh.at[didx_all.at[j]], ssem,
                             add=True)

            @pl.when(j >= SLAG)
            def _():
                drain_one()

            return carry

        lax.fori_loop(0, NCHUNK, step, 0)
        for _ in range(SLAG):
            drain_one()
        plsc.subcore_barrier()

        pltpu.sync_copy(acc_sh.at[pl.ds(sid * RPT, RPT)],
                        out_hbm.at[cid, pl.ds(sid * RPT, RPT)])

    return count_kernel


AGG_C = 40


@functools.lru_cache(maxsize=None)
def _make_agg(NP, E, D):
    NW = NC * NS
    C = AGG_C
    assert E % (NW * C) == 0
    EPW = E // NW
    NCHUNK = EPW // C
    RPT = NP // NS
    assert RPT % C == 0

    mesh = plsc.VectorSubcoreMesh(core_axis_name="c", subcore_axis_name="s")

    P = 5
    G = 2
    assert NCHUNK % P == 0 and G < P

    @functools.partial(
        pl.kernel,
        mesh=mesh,
        out_type=jax.ShapeDtypeStruct((NC, NP, D), jnp.float32),
        scratch_types=[
            pltpu.VMEM((NCHUNK, C), jnp.int32),
            pltpu.VMEM((NCHUNK, C), jnp.int32),
            pltpu.VMEM((P, C, D), jnp.float32),
            pltpu.VMEM_SHARED((NP, D), jnp.float32),
            pltpu.SemaphoreType.DMA,
            pltpu.SemaphoreType.DMA,
        ],
        compiler_params=pltpu.CompilerParams(use_tc_tiling_on_sc=False),
    )
    def agg_kernel(hs_hbm, src_hbm, dst_hbm, out_hbm,
                   sidx_all, didx_all, rows, acc_sh, gsem, ssem):
        cid = lax.axis_index("c")
        sid = lax.axis_index("s")
        wid = sid * NC + cid

        def fill(r, carry):
            for cc in range(D // L):
                rows[0, r, pl.ds(cc * L, L)] = jnp.zeros((L,), jnp.float32)
            return carry

        lax.fori_loop(0, C, fill, 0)

        pltpu.sync_copy(src_hbm.at[wid], sidx_all)
        pltpu.sync_copy(dst_hbm.at[wid], didx_all)
        for z in range(RPT // C):
            pltpu.sync_copy(rows.at[0],
                            acc_sh.at[pl.ds(sid * RPT + z * C, C)])
        plsc.subcore_barrier()

        def issue_gather(j, b):
            pltpu.async_copy(hs_hbm.at[sidx_all.at[j]], rows.at[b], gsem)

        def drain_gather():
            pltpu.make_async_copy(
                hs_hbm.at[sidx_all.at[0]], rows.at[0], gsem).wait()

        def drain_scatter():
            pltpu.make_async_copy(
                rows.at[0], acc_sh.at[didx_all.at[0]], ssem).wait()

        for b in range(G):
            issue_gather(b, b)

        def turn(g, carry):
            for b in range(P):
                t = g * P + b

                @pl.when(t >= G)
                def _():
                    drain_scatter()

                @pl.when(t + G < NCHUNK)
                def _():
                    issue_gather(t + G, (b + G) % P)

                drain_gather()
                pltpu.async_copy(rows.at[b], acc_sh.at[didx_all.at[t]],
                                 ssem, add=True)
            return carry

        lax.fori_loop(0, NCHUNK // P, turn, 0)
        for _ in range(G):
            drain_scatter()
        plsc.subcore_barrier()

        pltpu.sync_copy(acc_sh.at[pl.ds(sid * RPT, RPT)],
                        out_hbm.at[cid, pl.ds(sid * RPT, RPT)])

    return agg_kernel


def _dinv_block(cnt_ref):
    cnt = cnt_ref[0]
    deg = cnt[0] + cnt[1] + 1.0
    return lax.rsqrt(deg)


def _leaky(z):
    return jnp.where(z >= 0, z, 0.01 * z)


def _cnt_spec(R):
    return pl.BlockSpec((1, 2, R), lambda i: (i, 0, 0))


def _tc_pre_body(x_ref, w1_ref, wd_ref, bd_ref, cnt_ref, hs1_ref, ident_ref):
    dinv = _dinv_block(cnt_ref)
    x = x_ref[...]
    h1 = jnp.dot(x, w1_ref[...], preferred_element_type=jnp.float32)
    hs1_ref[...] = h1 * dinv[:, None]
    ident_ref[...] = (
        jnp.dot(x, wd_ref[...], preferred_element_type=jnp.float32)
        + bd_ref[...]
    )


def _tc_mid_body(agg_ref, hs1_ref, cnt_ref, b1_ref, w2_ref, hs2_ref):
    dinv = _dinv_block(cnt_ref)
    a = agg_ref[0] + agg_ref[1] + hs1_ref[...]
    o1 = _leaky(a * dinv[:, None] + b1_ref[...])
    h2 = jnp.dot(o1, w2_ref[...], preferred_element_type=jnp.float32)
    hs2_ref[...] = h2 * dinv[:, None]


def _tc_post_body(agg_ref, hs2_ref, cnt_ref, b2_ref, ident_ref, out_ref):
    dinv = _dinv_block(cnt_ref)
    a = agg_ref[0] + agg_ref[1] + hs2_ref[...]
    o2 = _leaky(a * dinv[:, None] + b2_ref[...])
    out_ref[...] = o2 + ident_ref[...]


def _row_spec(R, D):
    return pl.BlockSpec((R, D), lambda i: (i, 0))


def _part_spec(R, D):
    return pl.BlockSpec((2, R, D), lambda i: (0, i, 0))


def _full_spec(shape):
    nd = len(shape)
    return pl.BlockSpec(shape, lambda i: (0,) * nd)


def _tc_pre(x, W1, Wd, bd, cnt, R):
    NP, F = x.shape
    H = W1.shape[1]
    O = Wd.shape[1]
    grid = NP // R
    return pl.pallas_call(
        _tc_pre_body,
        grid=(grid,),
        in_specs=[
            _row_spec(R, F),
            _full_spec(W1.shape),
            _full_spec(Wd.shape),
            _full_spec((1, O)),
            _cnt_spec(R),
        ],
        out_specs=[_row_spec(R, H), _row_spec(R, O)],
        out_shape=[
            jax.ShapeDtypeStruct((NP, H), jnp.float32),
            jax.ShapeDtypeStruct((NP, O), jnp.float32),
        ],
    )(x, W1, Wd, bd.reshape(1, O), cnt)


def _tc_mid(agg1, hs1, cnt, b1, W2, R):
    NP, H = hs1.shape
    O = W2.shape[1]
    grid = NP // R
    return pl.pallas_call(
        _tc_mid_body,
        grid=(grid,),
        in_specs=[
            _part_spec(R, H),
            _row_spec(R, H),
            _cnt_spec(R),
            _full_spec((1, H)),
            _full_spec(W2.shape),
        ],
        out_specs=_row_spec(R, O),
        out_shape=jax.ShapeDtypeStruct((NP, O), jnp.float32),
    )(agg1, hs1, cnt, b1.reshape(1, H), W2)


def _tc_post(agg2, hs2, cnt, b2, ident, R):
    NP, O = hs2.shape
    grid = NP // R
    return pl.pallas_call(
        _tc_post_body,
        grid=(grid,),
        in_specs=[
            _part_spec(R, O),
            _row_spec(R, O),
            _cnt_spec(R),
            _full_spec((1, O)),
            _row_spec(R, O),
        ],
        out_specs=_row_spec(R, O),
        out_shape=jax.ShapeDtypeStruct((NP, O), jnp.float32),
    )(agg2, hs2, cnt, b2.reshape(1, O), ident)


@jax.jit
def kernel(x, edge_idx, W1, b1, W2, b2, Wd, bd):
    N, F = x.shape
    E = edge_idx.shape[1]
    H = W1.shape[1]
    O = W2.shape[1]
    R = 2000
    NP = _npad(N, 2048)

    NW = NC * NS
    src_r = edge_idx[0].reshape(NW, E // (NW * AGG_C), AGG_C)
    dst_r = edge_idx[1].reshape(NW, E // (NW * AGG_C), AGG_C)
    dst_c = edge_idx[1].reshape(NW, E // (NW * CNT_C), CNT_C)

    cnt_parts = _make_count(NP, E)(dst_c)
    cnt = cnt_parts[:, :N, 0]
    cnt = cnt.reshape(2, N // R, R).transpose(1, 0, 2)

    hs1, ident = _tc_pre(x, W1, Wd, bd, cnt, R)
    agg1 = _make_agg(NP, E, H)(hs1, src_r, dst_r)
    hs2 = _tc_mid(agg1, hs1, cnt, b1, W2, R)
    agg2 = _make_agg(NP, E, O)(hs2, src_r, dst_r)
    return _tc_post(agg2, hs2, cnt, b2, ident, R)


# speedup vs baseline: 38.5786x; 1.0814x over previous
import functools

import jax
import jax.numpy as jnp
from jax import lax
from jax.experimental import pallas as pl
from jax.experimental.pallas import tpu as pltpu
from jax.experimental.pallas import tpu_sc as plsc

NC = 2
NS = 16
L = 16


def _npad(N, R):
    NP = ((N + R - 1) // R) * R
    assert NP % (NS * 8) == 0
    return NP


CNT_C = 80


@functools.lru_cache(maxsize=None)
def _make_count(NP, E):
    NW = NC * NS
    C = CNT_C
    assert E % (NW * C) == 0
    EPW = E // NW
    NCHUNK = EPW // C
    RPT = NP // NS
    ZR = RPT // 5
    assert RPT % ZR == 0 and ZR >= C

    mesh = plsc.VectorSubcoreMesh(core_axis_name="c", subcore_axis_name="s")

    SLAG = 4

    @functools.partial(
        pl.kernel,
        mesh=mesh,
        out_type=jax.ShapeDtypeStruct((NC, NP, L), jnp.float32),
        scratch_types=[
            pltpu.VMEM((NCHUNK, C), jnp.int32),
            pltpu.VMEM((C, L), jnp.float32),
            pltpu.VMEM((ZR, L), jnp.float32),
            pltpu.VMEM_SHARED((NP, L), jnp.float32),
            pltpu.SemaphoreType.DMA,
        ],
        compiler_params=pltpu.CompilerParams(use_tc_tiling_on_sc=False),
    )
    def count_kernel(dst_hbm, out_hbm, didx_all, ones_v, zbuf, acc_sh, ssem):
        cid = lax.axis_index("c")
        sid = lax.axis_index("s")
        wid = sid * NC + cid

        def fill(r, carry):
            zbuf[r, :] = jnp.zeros((L,), jnp.float32)
            return carry

        lax.fori_loop(0, ZR, fill, 0)

        def fill_ones(r, carry):
            ones_v[r, :] = jnp.ones((L,), jnp.float32)
            return carry

        lax.fori_loop(0, C, fill_ones, 0)

        pltpu.sync_copy(dst_hbm.at[wid], didx_all)
        for z in range(RPT // ZR):
            pltpu.sync_copy(zbuf, acc_sh.at[pl.ds(sid * RPT + z * ZR, ZR)])
        plsc.subcore_barrier()

        def drain_one():
            pltpu.make_async_copy(
                ones_v, acc_sh.at[didx_all.at[0]], ssem).wait()

        def step(j, carry):
            pltpu.async_copy(ones_v, acc_sh.at[didx_all.at[j]], ssem,
                             add=True)

            @pl.when(j >= SLAG)
            def _():
                drain_one()

            return carry

        lax.fori_loop(0, NCHUNK, step, 0)
        for _ in range(SLAG):
            drain_one()
        plsc.subcore_barrier()

        pltpu.sync_copy(acc_sh.at[pl.ds(sid * RPT, RPT)],
                        out_hbm.at[cid, pl.ds(sid * RPT, RPT)])

    return count_kernel


def _agg_c(D):
    return 40 if D > 64 else 80


@functools.lru_cache(maxsize=None)
def _make_agg(NP, E, D):
    NW = NC * NS
    C = _agg_c(D)
    assert E % (NW * C) == 0
    EPW = E // NW
    NCHUNK = EPW // C
    RPT = NP // NS
    assert RPT % C == 0

    mesh = plsc.VectorSubcoreMesh(core_axis_name="c", subcore_axis_name="s")

    P = 5
    G = 2
    assert NCHUNK % P == 0 and G < P

    @functools.partial(
        pl.kernel,
        mesh=mesh,
        out_type=jax.ShapeDtypeStruct((NC, NP, D), jnp.float32),
        scratch_types=[
            pltpu.VMEM((NCHUNK, C), jnp.int32),
            pltpu.VMEM((NCHUNK, C), jnp.int32),
            pltpu.VMEM((P, C, D), jnp.float32),
            pltpu.VMEM_SHARED((NP, D), jnp.float32),
            pltpu.SemaphoreType.DMA,
            pltpu.SemaphoreType.DMA,
        ],
        compiler_params=pltpu.CompilerParams(use_tc_tiling_on_sc=False),
    )
    def agg_kernel(hs_hbm, src_hbm, dst_hbm, out_hbm,
                   sidx_all, didx_all, rows, acc_sh, gsem, ssem):
        cid = lax.axis_index("c")
        sid = lax.axis_index("s")
        wid = sid * NC + cid

        def fill(r, carry):
            for cc in range(D // L):
                rows[0, r, pl.ds(cc * L, L)] = jnp.zeros((L,), jnp.float32)
            return carry

        lax.fori_loop(0, C, fill, 0)

        pltpu.sync_copy(src_hbm.at[wid], sidx_all)
        pltpu.sync_copy(dst_hbm.at[wid], didx_all)
        for z in range(RPT // C):
            pltpu.sync_copy(rows.at[0],
                            acc_sh.at[pl.ds(sid * RPT + z * C, C)])
        plsc.subcore_barrier()

        def issue_gather(j, b):
            pltpu.async_copy(hs_hbm.at[sidx_all.at[j]], rows.at[b], gsem)

        def drain_gather():
            pltpu.make_async_copy(
                hs_hbm.at[sidx_all.at[0]], rows.at[0], gsem).wait()

        def drain_scatter():
            pltpu.make_async_copy(
                rows.at[0], acc_sh.at[didx_all.at[0]], ssem).wait()

        for b in range(G):
            issue_gather(b, b)

        def turn(g, carry):
            for b in range(P):
                t = g * P + b

                @pl.when(t >= G)
                def _():
                    drain_scatter()

                @pl.when(t + G < NCHUNK)
                def _():
                    issue_gather(t + G, (b + G) % P)

                drain_gather()
                pltpu.async_copy(rows.at[b], acc_sh.at[didx_all.at[t]],
                                 ssem, add=True)
            return carry

        lax.fori_loop(0, NCHUNK // P, turn, 0)
        for _ in range(G):
            drain_scatter()
        plsc.subcore_barrier()

        pltpu.sync_copy(acc_sh.at[pl.ds(sid * RPT, RPT)],
                        out_hbm.at[cid, pl.ds(sid * RPT, RPT)])

    return agg_kernel


def _dinv_block(cnt_ref):
    cnt = cnt_ref[0]
    deg = cnt[0] + cnt[1] + 1.0
    return lax.rsqrt(deg)


def _leaky(z):
    return jnp.where(z >= 0, z, 0.01 * z)


def _cnt_spec(R):
    return pl.BlockSpec((1, 2, R), lambda i: (i, 0, 0))


def _tc_pre_body(x_ref, w1_ref, wd_ref, bd_ref, cnt_ref, hs1_ref, ident_ref):
    dinv = _dinv_block(cnt_ref)
    x = x_ref[...]
    h1 = jnp.dot(x, w1_ref[...], preferred_element_type=jnp.float32)
    hs1_ref[...] = h1 * dinv[:, None]
    ident_ref[...] = (
        jnp.dot(x, wd_ref[...], preferred_element_type=jnp.float32)
        + bd_ref[...]
    )


def _tc_mid_body(agg_ref, hs1_ref, cnt_ref, b1_ref, w2_ref, hs2_ref):
    dinv = _dinv_block(cnt_ref)
    a = agg_ref[0] + agg_ref[1] + hs1_ref[...]
    o1 = _leaky(a * dinv[:, None] + b1_ref[...])
    h2 = jnp.dot(o1, w2_ref[...], preferred_element_type=jnp.float32)
    hs2_ref[...] = h2 * dinv[:, None]


def _tc_post_body(agg_ref, hs2_ref, cnt_ref, b2_ref, ident_ref, out_ref):
    dinv = _dinv_block(cnt_ref)
    a = agg_ref[0] + agg_ref[1] + hs2_ref[...]
    o2 = _leaky(a * dinv[:, None] + b2_ref[...])
    out_ref[...] = o2 + ident_ref[...]


def _row_spec(R, D):
    return pl.BlockSpec((R, D), lambda i: (i, 0))


def _part_spec(R, D):
    return pl.BlockSpec((2, R, D), lambda i: (0, i, 0))


def _full_spec(shape):
    nd = len(shape)
    return pl.BlockSpec(shape, lambda i: (0,) * nd)


def _tc_pre(x, W1, Wd, bd, cnt, R):
    NP, F = x.shape
    H = W1.shape[1]
    O = Wd.shape[1]
    grid = NP // R
    return pl.pallas_call(
        _tc_pre_body,
        grid=(grid,),
        in_specs=[
            _row_spec(R, F),
            _full_spec(W1.shape),
            _full_spec(Wd.shape),
            _full_spec((1, O)),
            _cnt_spec(R),
        ],
        out_specs=[_row_spec(R, H), _row_spec(R, O)],
        out_shape=[
            jax.ShapeDtypeStruct((NP, H), jnp.float32),
            jax.ShapeDtypeStruct((NP, O), jnp.float32),
        ],
    )(x, W1, Wd, bd.reshape(1, O), cnt)


def _tc_mid(agg1, hs1, cnt, b1, W2, R):
    NP, H = hs1.shape
    O = W2.shape[1]
    grid = NP // R
    return pl.pallas_call(
        _tc_mid_body,
        grid=(grid,),
        in_specs=[
            _part_spec(R, H),
            _row_spec(R, H),
            _cnt_spec(R),
            _full_spec((1, H)),
            _full_spec(W2.shape),
        ],
        out_specs=_row_spec(R, O),
        out_shape=jax.ShapeDtypeStruct((NP, O), jnp.float32),
    )(agg1, hs1, cnt, b1.reshape(1, H), W2)


def _tc_post(agg2, hs2, cnt, b2, ident, R):
    NP, O = hs2.shape
    grid = NP // R
    return pl.pallas_call(
        _tc_post_body,
        grid=(grid,),
        in_specs=[
            _part_spec(R, O),
            _row_spec(R, O),
            _cnt_spec(R),
            _full_spec((1, O)),
            _row_spec(R, O),
        ],
        out_specs=_row_spec(R, O),
        out_shape=jax.ShapeDtypeStruct((NP, O), jnp.float32),
    )(agg2, hs2, cnt, b2.reshape(1, O), ident)


@jax.jit
def kernel(x, edge_idx, W1, b1, W2, b2, Wd, bd):
    N, F = x.shape
    E = edge_idx.shape[1]
    H = W1.shape[1]
    O = W2.shape[1]
    R = 2000
    NP = _npad(N, 2048)

    NW = NC * NS
    C1, C2 = _agg_c(H), _agg_c(O)
    src_r1 = edge_idx[0].reshape(NW, E // (NW * C1), C1)
    dst_r1 = edge_idx[1].reshape(NW, E // (NW * C1), C1)
    src_r2 = edge_idx[0].reshape(NW, E // (NW * C2), C2)
    dst_r2 = edge_idx[1].reshape(NW, E // (NW * C2), C2)
    dst_c = edge_idx[1].reshape(NW, E // (NW * CNT_C), CNT_C)

    cnt_parts = _make_count(NP, E)(dst_c)
    cnt = cnt_parts[:, :N, 0]
    cnt = cnt.reshape(2, N // R, R).transpose(1, 0, 2)

    hs1, ident = _tc_pre(x, W1, Wd, bd, cnt, R)
    agg1 = _make_agg(NP, E, H)(hs1, src_r1, dst_r1)
    hs2 = _tc_mid(agg1, hs1, cnt, b1, W2, R)
    agg2 = _make_agg(NP, E, O)(hs2, src_r2, dst_r2)
    return _tc_post(agg2, hs2, cnt, b2, ident, R)


# speedup vs baseline: 43.3652x; 1.1241x over previous
import functools

import jax
import jax.numpy as jnp
from jax import lax
from jax.experimental import pallas as pl
from jax.experimental.pallas import tpu as pltpu
from jax.experimental.pallas import tpu_sc as plsc

NC = 2
NS = 16
L = 16


def _npad(N, R):
    NP = ((N + R - 1) // R) * R
    assert NP % (NS * 8) == 0
    return NP


CNT_C = 80


@functools.lru_cache(maxsize=None)
def _make_count(NP, E):
    NW = NC * NS
    C = CNT_C
    assert E % (NW * C) == 0
    EPW = E // NW
    NCHUNK = EPW // C
    RPT = NP // NS
    ZR = RPT // 5
    assert RPT % ZR == 0 and ZR >= C

    mesh = plsc.VectorSubcoreMesh(core_axis_name="c", subcore_axis_name="s")

    SLAG = 4

    @functools.partial(
        pl.kernel,
        mesh=mesh,
        out_type=jax.ShapeDtypeStruct((NC, NP), jnp.float32),
        scratch_types=[
            pltpu.VMEM((NCHUNK, C), jnp.int32),
            pltpu.VMEM((C,), jnp.float32),
            pltpu.VMEM((RPT,), jnp.float32),
            pltpu.VMEM_SHARED((NP,), jnp.float32),
            pltpu.SemaphoreType.DMA,
        ],
        compiler_params=pltpu.CompilerParams(use_tc_tiling_on_sc=False),
    )
    def count_kernel(edges_hbm, out_hbm, didx_all, ones_v, zbuf, acc_sh,
                     ssem):
        cid = lax.axis_index("c")
        sid = lax.axis_index("s")
        wid = sid * NC + cid

        def fill(r, carry):
            zbuf[pl.ds(r * L, L)] = jnp.zeros((L,), jnp.float32)
            return carry

        lax.fori_loop(0, RPT // L, fill, 0)

        def fill_ones(r, carry):
            ones_v[pl.ds(r * L, L)] = jnp.ones((L,), jnp.float32)
            return carry

        lax.fori_loop(0, C // L, fill_ones, 0)

        pltpu.sync_copy(edges_hbm.at[1, wid], didx_all)
        pltpu.sync_copy(zbuf, acc_sh.at[pl.ds(sid * RPT, RPT)])
        plsc.subcore_barrier()

        def drain_one():
            pltpu.make_async_copy(
                ones_v, acc_sh.at[didx_all.at[0]], ssem).wait()

        def step(j, carry):
            pltpu.async_copy(ones_v, acc_sh.at[didx_all.at[j]], ssem,
                             add=True)

            @pl.when(j >= SLAG)
            def _():
                drain_one()

            return carry

        lax.fori_loop(0, NCHUNK, step, 0)
        for _ in range(SLAG):
            drain_one()
        plsc.subcore_barrier()

        pltpu.sync_copy(acc_sh.at[pl.ds(sid * RPT, RPT)],
                        out_hbm.at[cid, pl.ds(sid * RPT, RPT)])

    return count_kernel


def _agg_c(D):
    return 40 if D > 64 else 80


@functools.lru_cache(maxsize=None)
def _make_agg(NP, E, D):
    NW = NC * NS
    C = _agg_c(D)
    assert E % (NW * C) == 0
    EPW = E // NW
    NCHUNK = EPW // C
    RPT = NP // NS
    assert RPT % C == 0

    mesh = plsc.VectorSubcoreMesh(core_axis_name="c", subcore_axis_name="s")

    P = 5
    G = 2
    assert NCHUNK % P == 0 and G < P

    @functools.partial(
        pl.kernel,
        mesh=mesh,
        out_type=jax.ShapeDtypeStruct((NC, NP, D), jnp.float32),
        scratch_types=[
            pltpu.VMEM((NCHUNK, C), jnp.int32),
            pltpu.VMEM((NCHUNK, C), jnp.int32),
            pltpu.VMEM((P, C, D), jnp.float32),
            pltpu.VMEM_SHARED((NP, D), jnp.float32),
            pltpu.SemaphoreType.DMA,
            pltpu.SemaphoreType.DMA,
        ],
        compiler_params=pltpu.CompilerParams(use_tc_tiling_on_sc=False),
    )
    def agg_kernel(hs_hbm, edges_hbm, out_hbm,
                   sidx_all, didx_all, rows, acc_sh, gsem, ssem):
        cid = lax.axis_index("c")
        sid = lax.axis_index("s")
        wid = sid * NC + cid

        def fill(r, carry):
            for cc in range(D // L):
                rows[0, r, pl.ds(cc * L, L)] = jnp.zeros((L,), jnp.float32)
            return carry

        lax.fori_loop(0, C, fill, 0)

        pltpu.sync_copy(edges_hbm.at[0, wid], sidx_all)
        pltpu.sync_copy(edges_hbm.at[1, wid], didx_all)
        for z in range(RPT // C):
            pltpu.sync_copy(rows.at[0],
                            acc_sh.at[pl.ds(sid * RPT + z * C, C)])
        plsc.subcore_barrier()

        def issue_gather(j, b):
            pltpu.async_copy(hs_hbm.at[sidx_all.at[j]], rows.at[b], gsem)

        def drain_gather():
            pltpu.make_async_copy(
                hs_hbm.at[sidx_all.at[0]], rows.at[0], gsem).wait()

        def drain_scatter():
            pltpu.make_async_copy(
                rows.at[0], acc_sh.at[didx_all.at[0]], ssem).wait()

        for b in range(G):
            issue_gather(b, b)

        def turn(g, carry):
            for b in range(P):
                t = g * P + b

                @pl.when(t >= G)
                def _():
                    drain_scatter()

                @pl.when(t + G < NCHUNK)
                def _():
                    issue_gather(t + G, (b + G) % P)

                drain_gather()
                pltpu.async_copy(rows.at[b], acc_sh.at[didx_all.at[t]],
                                 ssem, add=True)
            return carry

        lax.fori_loop(0, NCHUNK // P, turn, 0)
        for _ in range(G):
            drain_scatter()
        plsc.subcore_barrier()

        pltpu.sync_copy(acc_sh.at[pl.ds(sid * RPT, RPT)],
                        out_hbm.at[cid, pl.ds(sid * RPT, RPT)])

    return agg_kernel


def _dinv_block(cnt_ref):
    cnt = cnt_ref[0]
    deg = cnt[0] + cnt[1] + 1.0
    return lax.rsqrt(deg)


def _leaky(z):
    return jnp.where(z >= 0, z, 0.01 * z)


def _cnt_spec(R):
    return pl.BlockSpec((1, 2, R), lambda i: (i, 0, 0))


def _tc_pre_body(x_ref, w1_ref, wd_ref, bd_ref, cnt_ref, hs1_ref, ident_ref):
    dinv = _dinv_block(cnt_ref)
    x = x_ref[...]
    h1 = jnp.dot(x, w1_ref[...], preferred_element_type=jnp.float32)
    hs1_ref[...] = h1 * dinv[:, None]
    ident_ref[...] = (
        jnp.dot(x, wd_ref[...], preferred_element_type=jnp.float32)
        + bd_ref[...]
    )


def _tc_mid_body(agg_ref, hs1_ref, cnt_ref, b1_ref, w2_ref, hs2_ref):
    dinv = _dinv_block(cnt_ref)
    a = agg_ref[0] + agg_ref[1] + hs1_ref[...]
    o1 = _leaky(a * dinv[:, None] + b1_ref[...])
    h2 = jnp.dot(o1, w2_ref[...], preferred_element_type=jnp.float32)
    hs2_ref[...] = h2 * dinv[:, None]


def _tc_post_body(agg_ref, hs2_ref, cnt_ref, b2_ref, ident_ref, out_ref):
    dinv = _dinv_block(cnt_ref)
    a = agg_ref[0] + agg_ref[1] + hs2_ref[...]
    o2 = _leaky(a * dinv[:, None] + b2_ref[...])
    out_ref[...] = o2 + ident_ref[...]


def _row_spec(R, D):
    return pl.BlockSpec((R, D), lambda i: (i, 0))


def _part_spec(R, D):
    return pl.BlockSpec((2, R, D), lambda i: (0, i, 0))


def _full_spec(shape):
    nd = len(shape)
    return pl.BlockSpec(shape, lambda i: (0,) * nd)


def _tc_pre(x, W1, Wd, bd, cnt, R):
    NP, F = x.shape
    H = W1.shape[1]
    O = Wd.shape[1]
    grid = NP // R
    return pl.pallas_call(
        _tc_pre_body,
        grid=(grid,),
        in_specs=[
            _row_spec(R, F),
            _full_spec(W1.shape),
            _full_spec(Wd.shape),
            _full_spec((1, O)),
            _cnt_spec(R),
        ],
        out_specs=[_row_spec(R, H), _row_spec(R, O)],
        out_shape=[
            jax.ShapeDtypeStruct((NP, H), jnp.float32),
            jax.ShapeDtypeStruct((NP, O), jnp.float32),
        ],
    )(x, W1, Wd, bd.reshape(1, O), cnt)


def _tc_mid(agg1, hs1, cnt, b1, W2, R):
    NP, H = hs1.shape
    O = W2.shape[1]
    grid = NP // R
    return pl.pallas_call(
        _tc_mid_body,
        grid=(grid,),
        in_specs=[
            _part_spec(R, H),
            _row_spec(R, H),
            _cnt_spec(R),
            _full_spec((1, H)),
            _full_spec(W2.shape),
        ],
        out_specs=_row_spec(R, O),
        out_shape=jax.ShapeDtypeStruct((NP, O), jnp.float32),
    )(agg1, hs1, cnt, b1.reshape(1, H), W2)


def _tc_post(agg2, hs2, cnt, b2, ident, R):
    NP, O = hs2.shape
    grid = NP // R
    return pl.pallas_call(
        _tc_post_body,
        grid=(grid,),
        in_specs=[
            _part_spec(R, O),
            _row_spec(R, O),
            _cnt_spec(R),
            _full_spec((1, O)),
            _row_spec(R, O),
        ],
        out_specs=_row_spec(R, O),
        out_shape=jax.ShapeDtypeStruct((NP, O), jnp.float32),
    )(agg2, hs2, cnt, b2.reshape(1, O), ident)


@jax.jit
def kernel(x, edge_idx, W1, b1, W2, b2, Wd, bd):
    N, F = x.shape
    E = edge_idx.shape[1]
    H = W1.shape[1]
    O = W2.shape[1]
    R = 2000
    NP = _npad(N, 2048)

    NW = NC * NS
    C1, C2 = _agg_c(H), _agg_c(O)
    e3_1 = edge_idx.reshape(2, NW, E // (NW * C1), C1)
    e3_2 = edge_idx.reshape(2, NW, E // (NW * C2), C2)
    e3_c = edge_idx.reshape(2, NW, E // (NW * CNT_C), CNT_C)

    cnt_parts = _make_count(NP, E)(e3_c)
    cnt = cnt_parts[:, :N]
    cnt = cnt.reshape(2, N // R, R).transpose(1, 0, 2)

    hs1, ident = _tc_pre(x, W1, Wd, bd, cnt, R)
    agg1 = _make_agg(NP, E, H)(hs1, e3_1)
    hs2 = _tc_mid(agg1, hs1, cnt, b1, W2, R)
    agg2 = _make_agg(NP, E, O)(hs2, e3_2)
    return _tc_post(agg2, hs2, cnt, b2, ident, R)
